# baseline (speedup 1.0000x reference)
import jax
import jax.numpy as jnp
from jax import lax
from jax.experimental import pallas as pl
from jax.experimental.pallas import tpu as pltpu

B = 128
N = 32768
BLOCK_ROWS = 32
GRID = B // BLOCK_ROWS
CHUNK = 4096

ALPHA = 0.75
GAMMA = 2.0
HNM_RATIO = 10


def _block_kernel(pred_ref, target_ref, out_ref, hard_scratch):
    i = pl.program_id(0)

    acc_fr = jnp.zeros((BLOCK_ROWS, CHUNK), jnp.float32)
    acc_tfr = jnp.zeros((BLOCK_ROWS, CHUNK), jnp.float32)
    acc_t = jnp.zeros((BLOCK_ROWS, CHUNK), jnp.float32)
    for c in range(N // CHUNK):
        sl = pl.ds(c * CHUNK, CHUNK)
        tc = target_ref[:, sl]
        pc = pred_ref[:, sl]
        qc = jnp.clip(jnp.where(tc >= 0.5, pc, 1.0 - pc), 1e-06,
                      1.0 - 1e-06)
        mc = 1.0 - qc
        frc = mc * mc * jnp.log(qc)
        acc_fr = acc_fr + frc
        acc_tfr = acc_tfr + tc * frc
        acc_t = acc_t + tc

    tot_raw = jnp.sum(acc_fr, axis=1, keepdims=True)
    pos_raw = jnp.sum(acc_tfr, axis=1, keepdims=True)
    cnt_f = jnp.sum(acc_t, axis=1, keepdims=True)
    pos_cnt = cnt_f.astype(jnp.int32)
    pos_sum = -ALPHA * pos_raw
    neg_sum = -(1.0 - ALPHA) * (tot_raw - pos_raw)
    neg_cnt = N - pos_cnt
    n_hard = jnp.minimum(pos_cnt * HNM_RATIO, neg_cnt)

    easy = jnp.logical_or(n_hard == neg_cnt, n_hard <= 0)

    @pl.when(jnp.logical_not(jnp.all(easy)))
    def _search():
        t = target_ref[...]
        p = pred_ref[...]
        q = jnp.clip(jnp.where(t >= 0.5, p, 1.0 - p), 1e-06, 1.0 - 1e-06)
        one_m = 1.0 - q
        xm = jnp.where(t >= 0.5, -1.0, -(one_m * one_m * jnp.log(q)))

        def bit_step(j, res):
            trial = res | (1 << (30 - j))
            tval = lax.bitcast_convert_type(trial, jnp.float32)
            cnt = jnp.sum((xm >= tval).astype(jnp.int32), axis=1,
                          keepdims=True)
            return jnp.where(cnt >= n_hard, trial, res)

        res = lax.fori_loop(0, 31, bit_step,
                            jnp.zeros((BLOCK_ROWS, 1), jnp.int32))
        v = lax.bitcast_convert_type(res, jnp.float32)
        gt = xm > v
        cnt_gt = jnp.sum(gt.astype(jnp.int32), axis=1, keepdims=True)
        sum_gt = jnp.sum(jnp.where(gt, xm, 0.0), axis=1, keepdims=True)
        hard_scratch[...] = (1.0 - ALPHA) * (
            sum_gt + (n_hard - cnt_gt).astype(jnp.float32) * v)

    hard_sum = jnp.where(easy, jnp.where(n_hard > 0, neg_sum, 0.0),
                         hard_scratch[...])

    denom = jnp.maximum((pos_cnt + n_hard).astype(jnp.float32), 1.0)
    mean_pos = (pos_sum + hard_sum) / denom
    mean_neg = neg_sum / jnp.maximum(neg_cnt.astype(jnp.float32), 1.0)
    out_b = jnp.where(pos_cnt > 0, mean_pos, mean_neg)

    @pl.when(i == 0)
    def _init():
        out_ref[...] = jnp.zeros_like(out_ref)

    out_ref[...] += jnp.sum(out_b, keepdims=True) * (1.0 / B)


def kernel(pred, target):
    out = pl.pallas_call(
        _block_kernel,
        grid=(GRID,),
        in_specs=[
            pl.BlockSpec((BLOCK_ROWS, N), lambda i: (i, 0)),
            pl.BlockSpec((BLOCK_ROWS, N), lambda i: (i, 0)),
        ],
        out_specs=pl.BlockSpec((1, 1), lambda i: (0, 0)),
        out_shape=jax.ShapeDtypeStruct((1, 1), jnp.float32),
        scratch_shapes=[pltpu.VMEM((BLOCK_ROWS, 1), jnp.float32)],
    )(pred, target)
    return out[0, 0]


# speedup vs baseline: 1.0683x; 1.0683x over previous
import jax
import jax.numpy as jnp
from jax import lax
from jax.experimental import pallas as pl
from jax.experimental.pallas import tpu as pltpu

B = 128
N = 32768
BLOCK_ROWS = 32
GRID = B // BLOCK_ROWS
CHUNK = 1024

ALPHA = 0.75
GAMMA = 2.0
HNM_RATIO = 10


def _block_kernel(pred_ref, target_ref, out_ref, hard_scratch):
    i = pl.program_id(0)

    acc_fr = jnp.zeros((BLOCK_ROWS, CHUNK), jnp.float32)
    acc_tfr = jnp.zeros((BLOCK_ROWS, CHUNK), jnp.float32)
    acc_t = jnp.zeros((BLOCK_ROWS, CHUNK), jnp.float32)
    for c in range(N // CHUNK):
        sl = pl.ds(c * CHUNK, CHUNK)
        tc = target_ref[:, sl]
        pc = pred_ref[:, sl]
        qc = jnp.clip(jnp.where(tc >= 0.5, pc, 1.0 - pc), 1e-06,
                      1.0 - 1e-06)
        mc = 1.0 - qc
        frc = mc * mc * jnp.log(qc)
        acc_fr = acc_fr + frc
        acc_tfr = acc_tfr + tc * frc
        acc_t = acc_t + tc

    tot_raw = jnp.sum(acc_fr, axis=1, keepdims=True)
    pos_raw = jnp.sum(acc_tfr, axis=1, keepdims=True)
    cnt_f = jnp.sum(acc_t, axis=1, keepdims=True)
    pos_cnt = cnt_f.astype(jnp.int32)
    pos_sum = -ALPHA * pos_raw
    neg_sum = -(1.0 - ALPHA) * (tot_raw - pos_raw)
    neg_cnt = N - pos_cnt
    n_hard = jnp.minimum(pos_cnt * HNM_RATIO, neg_cnt)

    easy = jnp.logical_or(n_hard == neg_cnt, n_hard <= 0)

    @pl.when(jnp.logical_not(jnp.all(easy)))
    def _search():
        t = target_ref[...]
        p = pred_ref[...]
        q = jnp.clip(jnp.where(t >= 0.5, p, 1.0 - p), 1e-06, 1.0 - 1e-06)
        one_m = 1.0 - q
        xm = jnp.where(t >= 0.5, -1.0, -(one_m * one_m * jnp.log(q)))

        def bit_step(j, res):
            trial = res | (1 << (30 - j))
            tval = lax.bitcast_convert_type(trial, jnp.float32)
            cnt = jnp.sum((xm >= tval).astype(jnp.int32), axis=1,
                          keepdims=True)
            return jnp.where(cnt >= n_hard, trial, res)

        res = lax.fori_loop(0, 31, bit_step,
                            jnp.zeros((BLOCK_ROWS, 1), jnp.int32))
        v = lax.bitcast_convert_type(res, jnp.float32)
        gt = xm > v
        cnt_gt = jnp.sum(gt.astype(jnp.int32), axis=1, keepdims=True)
        sum_gt = jnp.sum(jnp.where(gt, xm, 0.0), axis=1, keepdims=True)
        hard_scratch[...] = (1.0 - ALPHA) * (
            sum_gt + (n_hard - cnt_gt).astype(jnp.float32) * v)

    hard_sum = jnp.where(easy, jnp.where(n_hard > 0, neg_sum, 0.0),
                         hard_scratch[...])

    denom = jnp.maximum((pos_cnt + n_hard).astype(jnp.float32), 1.0)
    mean_pos = (pos_sum + hard_sum) / denom
    mean_neg = neg_sum / jnp.maximum(neg_cnt.astype(jnp.float32), 1.0)
    out_b = jnp.where(pos_cnt > 0, mean_pos, mean_neg)

    @pl.when(i == 0)
    def _init():
        out_ref[...] = jnp.zeros_like(out_ref)

    out_ref[...] += jnp.sum(out_b, keepdims=True) * (1.0 / B)


def kernel(pred, target):
    out = pl.pallas_call(
        _block_kernel,
        grid=(GRID,),
        in_specs=[
            pl.BlockSpec((BLOCK_ROWS, N), lambda i: (i, 0)),
            pl.BlockSpec((BLOCK_ROWS, N), lambda i: (i, 0)),
        ],
        out_specs=pl.BlockSpec((1, 1), lambda i: (0, 0)),
        out_shape=jax.ShapeDtypeStruct((1, 1), jnp.float32),
        scratch_shapes=[pltpu.VMEM((BLOCK_ROWS, 1), jnp.float32)],
    )(pred, target)
    return out[0, 0]


# speedup vs baseline: 1.0712x; 1.0027x over previous
import jax
import jax.numpy as jnp
from jax import lax
from jax.experimental import pallas as pl
from jax.experimental.pallas import tpu as pltpu

B = 128
N = 32768
BLOCK_ROWS = 32
GRID = B // BLOCK_ROWS
CHUNK = 512

ALPHA = 0.75
GAMMA = 2.0
HNM_RATIO = 10


def _block_kernel(pred_ref, target_ref, out_ref, hard_scratch):
    i = pl.program_id(0)

    acc_fr = jnp.zeros((BLOCK_ROWS, CHUNK), jnp.float32)
    acc_tfr = jnp.zeros((BLOCK_ROWS, CHUNK), jnp.float32)
    acc_t = jnp.zeros((BLOCK_ROWS, CHUNK), jnp.float32)
    for c in range(N // CHUNK):
        sl = pl.ds(c * CHUNK, CHUNK)
        tc = target_ref[:, sl]
        pc = pred_ref[:, sl]
        qc = jnp.clip(jnp.where(tc >= 0.5, pc, 1.0 - pc), 1e-06,
                      1.0 - 1e-06)
        mc = 1.0 - qc
        frc = mc * mc * jnp.log(qc)
        acc_fr = acc_fr + frc
        acc_tfr = acc_tfr + tc * frc
        acc_t = acc_t + tc

    tot_raw = jnp.sum(acc_fr, axis=1, keepdims=True)
    pos_raw = jnp.sum(acc_tfr, axis=1, keepdims=True)
    cnt_f = jnp.sum(acc_t, axis=1, keepdims=True)
    pos_cnt = cnt_f.astype(jnp.int32)
    pos_sum = -ALPHA * pos_raw
    neg_sum = -(1.0 - ALPHA) * (tot_raw - pos_raw)
    neg_cnt = N - pos_cnt
    n_hard = jnp.minimum(pos_cnt * HNM_RATIO, neg_cnt)

    easy = jnp.logical_or(n_hard == neg_cnt, n_hard <= 0)

    @pl.when(jnp.logical_not(jnp.all(easy)))
    def _search():
        t = target_ref[...]
        p = pred_ref[...]
        q = jnp.clip(jnp.where(t >= 0.5, p, 1.0 - p), 1e-06, 1.0 - 1e-06)
        one_m = 1.0 - q
        xm = jnp.where(t >= 0.5, -1.0, -(one_m * one_m * jnp.log(q)))

        def bit_step(j, res):
            trial = res | (1 << (30 - j))
            tval = lax.bitcast_convert_type(trial, jnp.float32)
            cnt = jnp.sum((xm >= tval).astype(jnp.int32), axis=1,
                          keepdims=True)
            return jnp.where(cnt >= n_hard, trial, res)

        res = lax.fori_loop(0, 31, bit_step,
                            jnp.zeros((BLOCK_ROWS, 1), jnp.int32))
        v = lax.bitcast_convert_type(res, jnp.float32)
        gt = xm > v
        cnt_gt = jnp.sum(gt.astype(jnp.int32), axis=1, keepdims=True)
        sum_gt = jnp.sum(jnp.where(gt, xm, 0.0), axis=1, keepdims=True)
        hard_scratch[...] = (1.0 - ALPHA) * (
            sum_gt + (n_hard - cnt_gt).astype(jnp.float32) * v)

    hard_sum = jnp.where(easy, jnp.where(n_hard > 0, neg_sum, 0.0),
                         hard_scratch[...])

    denom = jnp.maximum((pos_cnt + n_hard).astype(jnp.float32), 1.0)
    mean_pos = (pos_sum + hard_sum) / denom
    mean_neg = neg_sum / jnp.maximum(neg_cnt.astype(jnp.float32), 1.0)
    out_b = jnp.where(pos_cnt > 0, mean_pos, mean_neg)

    @pl.when(i == 0)
    def _init():
        out_ref[...] = jnp.zeros_like(out_ref)

    out_ref[...] += jnp.sum(out_b, keepdims=True) * (1.0 / B)


def kernel(pred, target):
    out = pl.pallas_call(
        _block_kernel,
        grid=(GRID,),
        in_specs=[
            pl.BlockSpec((BLOCK_ROWS, N), lambda i: (i, 0)),
            pl.BlockSpec((BLOCK_ROWS, N), lambda i: (i, 0)),
        ],
        out_specs=pl.BlockSpec((1, 1), lambda i: (0, 0)),
        out_shape=jax.ShapeDtypeStruct((1, 1), jnp.float32),
        scratch_shapes=[pltpu.VMEM((BLOCK_ROWS, 1), jnp.float32)],
    )(pred, target)
    return out[0, 0]
